# baseline (speedup 1.0000x reference)
import functools

import jax
import jax.numpy as jnp
from jax import lax
from jax.experimental import pallas as pl
from jax.experimental.pallas import tpu as pltpu
from jax.experimental.pallas import tpu_sc as plsc

NC = 2
NS = 16
NW = NC * NS
CHUNK = 128
NBUF = 4


def _mesh():
    return plsc.VectorSubcoreMesh(core_axis_name="c", subcore_axis_name="s")


def _make_degree_kernel(n_pad, k):
    n2 = 2 * n_pad
    zchunk = n2 // 16
    assert n2 % zchunk == 0 and zchunk % 16 == 0

    @functools.partial(
        pl.kernel,
        out_type=jax.ShapeDtypeStruct((NC, n2), jnp.float32),
        mesh=_mesh(),
        scratch_types=[
            pltpu.VMEM((k, CHUNK), jnp.int32),
            pltpu.VMEM((k, CHUNK), jnp.int32),
            pltpu.VMEM((2 * k, CHUNK), jnp.int32),
            pltpu.VMEM((zchunk,), jnp.float32),
            pltpu.VMEM((CHUNK,), jnp.float32),
            pltpu.VMEM_SHARED((n2,), jnp.float32),
            pltpu.SemaphoreType.DMA,
        ],
        compiler_params=pltpu.CompilerParams(use_tc_tiling_on_sc=False),
    )
    def deg_kernel(src_hbm, dst_hbm, out_hbm,
                   src_v, dst_v, didx_v, zbuf, ones_v, acc, sem):
        cid = lax.axis_index("c")
        sid = lax.axis_index("s")
        wid = sid * NC + cid

        @pl.when(sid == 0)
        def _():
            def zfill(i, carry):
                zbuf[pl.ds(i * 16, 16)] = jnp.zeros((16,), jnp.float32)
                return carry
            lax.fori_loop(0, zchunk // 16, zfill, 0)
            for i in range(n2 // zchunk):
                pltpu.sync_copy(zbuf, acc.at[pl.ds(i * zchunk, zchunk)])

        def ofill(i, carry):
            ones_v[pl.ds(i * 16, 16)] = jnp.ones((16,), jnp.float32)
            return carry
        lax.fori_loop(0, CHUNK // 16, ofill, 0)

        pltpu.sync_copy(src_hbm.at[wid], src_v)
        pltpu.sync_copy(dst_hbm.at[wid], dst_v)

        def build(j, carry):
            for t in range(CHUNK // 16):
                sl = pl.ds(t * 16, 16)
                didx_v[j, sl] = src_v[j, sl] * 2
                didx_v[j + k, sl] = dst_v[j, sl] * 2 + 1
            return carry
        lax.fori_loop(0, k, build, 0)
        plsc.subcore_barrier()

        def fire(j, carry):
            pltpu.async_copy(ones_v, acc.at[didx_v.at[j]], sem, add=True)
            return carry
        lax.fori_loop(0, 2 * k, fire, 0)

        def drain(j, carry):
            pltpu.make_async_copy(ones_v, acc.at[didx_v.at[j]], sem).wait()
            return carry
        lax.fori_loop(0, 2 * k, drain, 0)
        plsc.subcore_barrier()

        @pl.when(sid == 0)
        def _():
            pltpu.sync_copy(acc, out_hbm.at[cid])

    return deg_kernel


def _make_agg_kernel(n_pad, k, f):
    rows_per_tile = n_pad // NS
    assert rows_per_tile % 8 == 0

    @functools.partial(
        pl.kernel,
        out_type=jax.ShapeDtypeStruct((NC, n_pad, f), jnp.float32),
        mesh=_mesh(),
        scratch_types=[
            pltpu.VMEM((k, CHUNK), jnp.int32),
            pltpu.VMEM((k, CHUNK), jnp.int32),
            pltpu.VMEM((NBUF, CHUNK, f), jnp.float32),
            pltpu.VMEM((rows_per_tile, f), jnp.float32),
            pltpu.VMEM_SHARED((n_pad, f), jnp.float32),
            pltpu.SemaphoreType.DMA((NBUF,)),
            pltpu.SemaphoreType.DMA((NBUF,)),
        ],
        compiler_params=pltpu.CompilerParams(use_tc_tiling_on_sc=False),
    )
    def agg_kernel(h_hbm, src_hbm, dst_hbm, out_hbm,
                   src_v, dst_v, buf, zbuf, acc, gsem, ssem):
        cid = lax.axis_index("c")
        sid = lax.axis_index("s")
        wid = sid * NC + cid

        def zfill(i, carry):
            zbuf[i, :] = jnp.zeros((f,), jnp.float32)
            return carry
        lax.fori_loop(0, rows_per_tile, zfill, 0)
        pltpu.sync_copy(zbuf, acc.at[pl.ds(sid * rows_per_tile, rows_per_tile)])

        pltpu.sync_copy(src_hbm.at[wid], src_v)
        pltpu.sync_copy(dst_hbm.at[wid], dst_v)
        plsc.subcore_barrier()

        pltpu.async_copy(h_hbm.at[src_v.at[0]], buf.at[0], gsem.at[0])

        def body(j, carry):
            @pl.when(j + 1 < k)
            def _():
                nslot = lax.rem(j + 1, NBUF)
                pltpu.async_copy(h_hbm.at[src_v.at[j + 1]],
                                 buf.at[nslot], gsem.at[nslot])
            slot = lax.rem(j, NBUF)
            pltpu.make_async_copy(h_hbm.at[src_v.at[j]],
                                  buf.at[slot], gsem.at[slot]).wait()
            pltpu.sync_copy(buf.at[slot], acc.at[dst_v.at[j]], add=True)
            return carry

        lax.fori_loop(0, k, body, 0)
        plsc.subcore_barrier()

        pltpu.sync_copy(
            acc.at[pl.ds(sid * rows_per_tile, rows_per_tile)],
            out_hbm.at[cid, pl.ds(sid * rows_per_tile, rows_per_tile)])

    return agg_kernel


def _tc_matmul(x, w, bm=2000):
    n, kin = x.shape
    f = w.shape[1]

    def mm_kernel(x_ref, w_ref, o_ref):
        o_ref[...] = lax.dot_general(
            x_ref[...], w_ref[...], (((1,), (0,)), ((), ())),
            preferred_element_type=jnp.float32)

    return pl.pallas_call(
        mm_kernel,
        grid=(n // bm,),
        in_specs=[pl.BlockSpec((bm, kin), lambda i: (i, 0)),
                  pl.BlockSpec((kin, f), lambda i: (0, 0))],
        out_specs=pl.BlockSpec((bm, f), lambda i: (i, 0)),
        out_shape=jax.ShapeDtypeStruct((n, f), jnp.float32),
    )(x, w)


def _tc_scale_by_dsqo(y, deg, n_pad):
    n, f = y.shape

    def body(y_ref, d_ref, o_ref):
        d = d_ref[0, :n, :] + d_ref[1, :n, :]
        dsqo = lax.rsqrt(jnp.maximum(d[:, 0:1], 1.0))
        o_ref[:n, :] = y_ref[...] * dsqo
        o_ref[n:, :] = jnp.zeros((n_pad - n, f), jnp.float32)

    return pl.pallas_call(
        body,
        out_shape=jax.ShapeDtypeStruct((n_pad, f), jnp.float32),
    )(y, deg)


def _tc_relu_rescale(agg_parts, deg, b1, n):
    _, n_pad, f = agg_parts.shape

    def body(a_ref, d_ref, b_ref, o_ref):
        a = a_ref[0, :n, :] + a_ref[1, :n, :]
        d = d_ref[0, :n, :] + d_ref[1, :n, :]
        dsqo = lax.rsqrt(jnp.maximum(d[:, 0:1], 1.0))
        dsqi = lax.rsqrt(jnp.maximum(d[:, 1:2], 1.0))
        h = jnp.maximum(a * dsqi + b_ref[...], 0.0)
        o_ref[:n, :] = h * dsqo
        o_ref[n:, :] = jnp.zeros((n_pad - n, f), jnp.float32)

    return pl.pallas_call(
        body,
        out_shape=jax.ShapeDtypeStruct((n_pad, f), jnp.float32),
    )(agg_parts, deg, b1.reshape(1, f))


def _tc_final(agg_parts, deg, w2, b2, n):
    _, n_pad, f = agg_parts.shape
    fo = w2.shape[1]

    def body(a_ref, d_ref, w_ref, b_ref, o_ref):
        a = a_ref[0, :n, :] + a_ref[1, :n, :]
        d = d_ref[0, :n, :] + d_ref[1, :n, :]
        dsqi = lax.rsqrt(jnp.maximum(d[:, 1:2], 1.0))
        h = a * dsqi
        o_ref[...] = lax.dot_general(
            h, w_ref[...], (((1,), (0,)), ((), ())),
            preferred_element_type=jnp.float32) + b_ref[...]

    return pl.pallas_call(
        body,
        out_shape=jax.ShapeDtypeStruct((n, fo), jnp.float32),
    )(agg_parts, deg, w2, b2.reshape(1, fo))


def kernel(features, edge_index, W1, b1, W2, b2):
    n, _ = features.shape
    e = edge_index.shape[1]
    n_pad = ((n + 8 * NS - 1) // (8 * NS)) * (8 * NS)
    trash = n_pad - 1

    k = -(-e // (NW * CHUNK))
    e_pad = NW * k * CHUNK
    pad = jnp.full((e_pad - e,), trash, jnp.int32)
    src_r = jnp.concatenate([edge_index[0], pad]).reshape(NW, k, CHUNK)
    dst_r = jnp.concatenate([edge_index[1], pad]).reshape(NW, k, CHUNK)

    deg = _make_degree_kernel(n_pad, k)(src_r, dst_r).reshape(NC, n_pad, 2)
    y = _tc_matmul(features, W1)
    h1 = _tc_scale_by_dsqo(y, deg, n_pad)

    agg16 = _make_agg_kernel(n_pad, k, 16)
    a1 = agg16(h1, src_r, dst_r)
    scaled = _tc_relu_rescale(a1, deg, b1, n)
    a2 = agg16(scaled, src_r, dst_r)
    return _tc_final(a2, deg, W2, b2, n)


# speedup vs baseline: 1.4013x; 1.4013x over previous
import functools

import jax
import jax.numpy as jnp
from jax import lax
from jax.experimental import pallas as pl
from jax.experimental.pallas import tpu as pltpu
from jax.experimental.pallas import tpu_sc as plsc

NC = 2
NS = 16
NW = NC * NS
CHUNK = 128
NBUF = 4


def _mesh():
    return plsc.VectorSubcoreMesh(core_axis_name="c", subcore_axis_name="s")


def _make_degree_kernel(n_pad, k):
    rows_per_tile = n_pad // NS
    assert rows_per_tile % 8 == 0

    @functools.partial(
        pl.kernel,
        out_type=jax.ShapeDtypeStruct((NC, 2, n_pad, 16), jnp.float32),
        mesh=_mesh(),
        scratch_types=[
            pltpu.VMEM((k, CHUNK), jnp.int32),
            pltpu.VMEM((k, CHUNK), jnp.int32),
            pltpu.VMEM((CHUNK, 16), jnp.float32),
            pltpu.VMEM((rows_per_tile, 16), jnp.float32),
            pltpu.VMEM_SHARED((n_pad, 16), jnp.float32),
            pltpu.VMEM_SHARED((n_pad, 16), jnp.float32),
            pltpu.SemaphoreType.DMA,
        ],
        compiler_params=pltpu.CompilerParams(use_tc_tiling_on_sc=False),
    )
    def deg_kernel(src_hbm, dst_hbm, out_hbm,
                   src_v, dst_v, ones_v, zbuf, acc_o, acc_i, sem):
        cid = lax.axis_index("c")
        sid = lax.axis_index("s")
        wid = sid * NC + cid
        row0 = sid * rows_per_tile

        def zfill(i, carry):
            zbuf[i, :] = jnp.zeros((16,), jnp.float32)
            return carry
        lax.fori_loop(0, rows_per_tile, zfill, 0)

        def ofill(i, carry):
            ones_v[i, :] = jnp.ones((16,), jnp.float32)
            return carry
        lax.fori_loop(0, CHUNK, ofill, 0)

        pltpu.sync_copy(zbuf, acc_o.at[pl.ds(row0, rows_per_tile)])
        pltpu.sync_copy(zbuf, acc_i.at[pl.ds(row0, rows_per_tile)])
        pltpu.sync_copy(src_hbm.at[wid], src_v)
        pltpu.sync_copy(dst_hbm.at[wid], dst_v)
        plsc.subcore_barrier()

        def fire(j, carry):
            pltpu.async_copy(ones_v, acc_o.at[src_v.at[j]], sem, add=True)
            pltpu.async_copy(ones_v, acc_i.at[dst_v.at[j]], sem, add=True)
            return carry
        lax.fori_loop(0, k, fire, 0)

        def drain(j, carry):
            pltpu.make_async_copy(ones_v, acc_o.at[src_v.at[j]], sem).wait()
            pltpu.make_async_copy(ones_v, acc_i.at[dst_v.at[j]], sem).wait()
            return carry
        lax.fori_loop(0, k, drain, 0)
        plsc.subcore_barrier()

        pltpu.sync_copy(acc_o.at[pl.ds(row0, rows_per_tile)],
                        out_hbm.at[cid, 0, pl.ds(row0, rows_per_tile)])
        pltpu.sync_copy(acc_i.at[pl.ds(row0, rows_per_tile)],
                        out_hbm.at[cid, 1, pl.ds(row0, rows_per_tile)])

    return deg_kernel


def _make_agg_kernel(n_pad, k, f):
    rows_per_tile = n_pad // NS
    assert rows_per_tile % 8 == 0

    @functools.partial(
        pl.kernel,
        out_type=jax.ShapeDtypeStruct((NC, n_pad, f), jnp.float32),
        mesh=_mesh(),
        scratch_types=[
            pltpu.VMEM((k, CHUNK), jnp.int32),
            pltpu.VMEM((k, CHUNK), jnp.int32),
            pltpu.VMEM((NBUF, CHUNK, f), jnp.float32),
            pltpu.VMEM((rows_per_tile, f), jnp.float32),
            pltpu.VMEM_SHARED((n_pad, f), jnp.float32),
            pltpu.SemaphoreType.DMA((NBUF,)),
        ],
        compiler_params=pltpu.CompilerParams(use_tc_tiling_on_sc=False),
    )
    def agg_kernel(h_hbm, src_hbm, dst_hbm, out_hbm,
                   src_v, dst_v, buf, zbuf, acc, gsem):
        cid = lax.axis_index("c")
        sid = lax.axis_index("s")
        wid = sid * NC + cid
        row0 = sid * rows_per_tile

        def zfill(i, carry):
            zbuf[i, :] = jnp.zeros((f,), jnp.float32)
            return carry
        lax.fori_loop(0, rows_per_tile, zfill, 0)
        pltpu.sync_copy(zbuf, acc.at[pl.ds(row0, rows_per_tile)])

        pltpu.sync_copy(src_hbm.at[wid], src_v)
        pltpu.sync_copy(dst_hbm.at[wid], dst_v)
        plsc.subcore_barrier()

        pltpu.async_copy(h_hbm.at[src_v.at[0]], buf.at[0], gsem.at[0])

        def body(j, carry):
            @pl.when(j + 1 < k)
            def _():
                nslot = lax.rem(j + 1, NBUF)
                pltpu.async_copy(h_hbm.at[src_v.at[j + 1]],
                                 buf.at[nslot], gsem.at[nslot])
            slot = lax.rem(j, NBUF)
            pltpu.make_async_copy(h_hbm.at[src_v.at[j]],
                                  buf.at[slot], gsem.at[slot]).wait()
            pltpu.sync_copy(buf.at[slot], acc.at[dst_v.at[j]], add=True)
            return carry

        lax.fori_loop(0, k, body, 0)
        plsc.subcore_barrier()

        pltpu.sync_copy(
            acc.at[pl.ds(row0, rows_per_tile)],
            out_hbm.at[cid, pl.ds(row0, rows_per_tile)])

    return agg_kernel


def _tc_matmul(x, w, n_pad, bm):
    n, kin = x.shape
    f = w.shape[1]
    assert n_pad % bm == 0 and bm % 8 == 0

    def body(x_ref, w_ref, o_ref):
        o_ref[...] = lax.dot_general(
            x_ref[...], w_ref[...], (((1,), (0,)), ((), ())),
            preferred_element_type=jnp.float32)

    return pl.pallas_call(
        body,
        grid=(n_pad // bm,),
        in_specs=[pl.BlockSpec((bm, kin), lambda i: (i, 0)),
                  pl.BlockSpec((kin, f), lambda i: (0, 0))],
        out_specs=pl.BlockSpec((bm, f), lambda i: (i, 0)),
        out_shape=jax.ShapeDtypeStruct((n_pad, f), jnp.float32),
    )(x, w)


def _tc_scale_flat(yf, degf):
    nf, _ = yf.shape

    def body(y_ref, d_ref, o_ref):
        d = d_ref[0, 0] + d_ref[1, 0]
        o_ref[...] = y_ref[...] * lax.rsqrt(jnp.maximum(d, 1.0))

    return pl.pallas_call(
        body,
        out_shape=jax.ShapeDtypeStruct((nf, 128), jnp.float32),
    )(yf, degf)


def _tc_relu_rescale(af, degf, b1):
    _, nf, _ = af.shape

    def body(a_ref, d_ref, b_ref, o_ref):
        a = a_ref[0] + a_ref[1]
        do = d_ref[0, 0] + d_ref[1, 0]
        di = d_ref[0, 1] + d_ref[1, 1]
        h = jnp.maximum(a * lax.rsqrt(jnp.maximum(di, 1.0)) + b_ref[...], 0.0)
        o_ref[...] = h * lax.rsqrt(jnp.maximum(do, 1.0))

    return pl.pallas_call(
        body,
        out_shape=jax.ShapeDtypeStruct((nf, 128), jnp.float32),
    )(af, degf, b1)


def _tc_final_flat(af, degf, w2bd, b2t):
    _, nf, _ = af.shape
    fo8 = w2bd.shape[1]

    def body(a_ref, d_ref, w_ref, b_ref, o_ref):
        a = a_ref[0] + a_ref[1]
        di = d_ref[0, 1] + d_ref[1, 1]
        hf = a * lax.rsqrt(jnp.maximum(di, 1.0))
        res = lax.dot_general(hf, w_ref[...], (((1,), (0,)), ((), ())),
                              preferred_element_type=jnp.float32)
        o_ref[...] = res + b_ref[...]

    return pl.pallas_call(
        body,
        out_shape=jax.ShapeDtypeStruct((nf, fo8), jnp.float32),
    )(af, degf, w2bd, b2t)


def kernel(features, edge_index, W1, b1, W2, b2):
    n, _ = features.shape
    e = edge_index.shape[1]
    n_pad = ((n + 8 * NS - 1) // (8 * NS)) * (8 * NS)
    nf = n_pad // 8
    trash = n_pad - 1

    k = -(-e // (NW * CHUNK))
    e_pad = NW * k * CHUNK
    pad = jnp.full((e_pad - e,), trash, jnp.int32)
    src_r = jnp.concatenate([edge_index[0], pad]).reshape(NW, k, CHUNK)
    dst_r = jnp.concatenate([edge_index[1], pad]).reshape(NW, k, CHUNK)

    deg = _make_degree_kernel(n_pad, k)(src_r, dst_r)
    degf = deg.reshape(NC, 2, nf, 128)
    y = _tc_matmul(features, W1, n_pad, bm=1264)
    h1f = _tc_scale_flat(y.reshape(nf, 128), degf)

    agg16 = _make_agg_kernel(n_pad, k, 16)
    b1t = jnp.tile(b1, 8).reshape(1, 128)
    a1 = agg16(h1f.reshape(n_pad, 16), src_r, dst_r)
    scaledf = _tc_relu_rescale(a1.reshape(NC, nf, 128), degf, b1t)
    a2 = agg16(scaledf.reshape(n_pad, 16), src_r, dst_r)

    fo = W2.shape[1]
    w2bd = jnp.kron(jnp.eye(8, dtype=jnp.float32), W2)
    b2t = jnp.tile(b2, 8).reshape(1, 8 * fo)
    res = _tc_final_flat(a2.reshape(NC, nf, 128), degf, w2bd, b2t)
    return res.reshape(n_pad, fo)[:n]


# speedup vs baseline: 1.7543x; 1.2519x over previous
import functools

import jax
import jax.numpy as jnp
from jax import lax
from jax.experimental import pallas as pl
from jax.experimental.pallas import tpu as pltpu
from jax.experimental.pallas import tpu_sc as plsc

NC = 2
NS = 16
NW = NC * NS
CHUNK = 128
NBUF = 4


def _mesh():
    return plsc.VectorSubcoreMesh(core_axis_name="c", subcore_axis_name="s")


def _make_degree_kernel(n_pad, k):
    rows_per_tile = n_pad // NS
    epw = k * CHUNK
    assert rows_per_tile % 8 == 0 and epw % 8 == 0

    @functools.partial(
        pl.kernel,
        out_type=jax.ShapeDtypeStruct((NC, 2, n_pad, 16), jnp.float32),
        mesh=_mesh(),
        scratch_types=[
            pltpu.VMEM((epw,), jnp.int32),
            pltpu.VMEM((epw,), jnp.int32),
            pltpu.VMEM((CHUNK, 16), jnp.float32),
            pltpu.VMEM((rows_per_tile, 16), jnp.float32),
            pltpu.VMEM_SHARED((n_pad, 16), jnp.float32),
            pltpu.VMEM_SHARED((n_pad, 16), jnp.float32),
            pltpu.SemaphoreType.DMA,
        ],
        compiler_params=pltpu.CompilerParams(use_tc_tiling_on_sc=False),
    )
    def deg_kernel(src_hbm, dst_hbm, out_hbm,
                   src_v, dst_v, ones_v, zbuf, acc_o, acc_i, sem):
        cid = lax.axis_index("c")
        sid = lax.axis_index("s")
        wid = sid * NC + cid
        row0 = sid * rows_per_tile

        def zfill(i, carry):
            zbuf[i, :] = jnp.zeros((16,), jnp.float32)
            return carry
        lax.fori_loop(0, rows_per_tile, zfill, 0)

        def ofill(i, carry):
            ones_v[i, :] = jnp.ones((16,), jnp.float32)
            return carry
        lax.fori_loop(0, CHUNK, ofill, 0)

        pltpu.sync_copy(zbuf, acc_o.at[pl.ds(row0, rows_per_tile)])
        pltpu.sync_copy(zbuf, acc_i.at[pl.ds(row0, rows_per_tile)])
        pltpu.sync_copy(src_hbm.at[pl.ds(wid * epw, epw)], src_v)
        pltpu.sync_copy(dst_hbm.at[pl.ds(wid * epw, epw)], dst_v)
        plsc.subcore_barrier()

        def fire(j, carry):
            off = pl.multiple_of(j * CHUNK, 8)
            pltpu.async_copy(ones_v, acc_o.at[src_v.at[pl.ds(off, CHUNK)]],
                             sem, add=True)
            pltpu.async_copy(ones_v, acc_i.at[dst_v.at[pl.ds(off, CHUNK)]],
                             sem, add=True)
            return carry
        lax.fori_loop(0, k, fire, 0)

        def drain(j, carry):
            off = pl.multiple_of(j * CHUNK, 8)
            pltpu.make_async_copy(
                ones_v, acc_o.at[src_v.at[pl.ds(off, CHUNK)]], sem).wait()
            pltpu.make_async_copy(
                ones_v, acc_i.at[dst_v.at[pl.ds(off, CHUNK)]], sem).wait()
            return carry
        lax.fori_loop(0, k, drain, 0)
        plsc.subcore_barrier()

        pltpu.sync_copy(acc_o.at[pl.ds(row0, rows_per_tile)],
                        out_hbm.at[cid, 0, pl.ds(row0, rows_per_tile)])
        pltpu.sync_copy(acc_i.at[pl.ds(row0, rows_per_tile)],
                        out_hbm.at[cid, 1, pl.ds(row0, rows_per_tile)])

    return deg_kernel


def _make_agg_kernel(n_pad, k, f):
    rows_per_tile = n_pad // NS
    epw = k * CHUNK
    assert rows_per_tile % 8 == 0 and epw % 8 == 0

    @functools.partial(
        pl.kernel,
        out_type=jax.ShapeDtypeStruct((NC, n_pad, f), jnp.float32),
        mesh=_mesh(),
        scratch_types=[
            pltpu.VMEM((epw,), jnp.int32),
            pltpu.VMEM((epw,), jnp.int32),
            pltpu.VMEM((NBUF, CHUNK, f), jnp.float32),
            pltpu.VMEM((rows_per_tile, f), jnp.float32),
            pltpu.VMEM_SHARED((n_pad, f), jnp.float32),
            pltpu.VMEM_SHARED((n_pad, f), jnp.float32),
            pltpu.SemaphoreType.DMA((NBUF,)),
        ],
        compiler_params=pltpu.CompilerParams(use_tc_tiling_on_sc=False),
    )
    def agg_kernel(h_hbm, src_hbm, dst_hbm, out_hbm,
                   src_v, dst_v, buf, zbuf, acc, h_spm, gsem):
        cid = lax.axis_index("c")
        sid = lax.axis_index("s")
        wid = sid * NC + cid
        row0 = sid * rows_per_tile

        def zfill(i, carry):
            zbuf[i, :] = jnp.zeros((f,), jnp.float32)
            return carry
        lax.fori_loop(0, rows_per_tile, zfill, 0)
        pltpu.sync_copy(zbuf, acc.at[pl.ds(row0, rows_per_tile)])

        pltpu.sync_copy(h_hbm.at[pl.ds(row0, rows_per_tile)],
                        h_spm.at[pl.ds(row0, rows_per_tile)])
        pltpu.sync_copy(src_hbm.at[pl.ds(wid * epw, epw)], src_v)
        pltpu.sync_copy(dst_hbm.at[pl.ds(wid * epw, epw)], dst_v)
        plsc.subcore_barrier()

        pltpu.async_copy(h_spm.at[src_v.at[pl.ds(0, CHUNK)]],
                         buf.at[0], gsem.at[0])

        def body(j, carry):
            @pl.when(j + 1 < k)
            def _():
                noff = pl.multiple_of((j + 1) * CHUNK, 8)
                nslot = lax.rem(j + 1, NBUF)
                pltpu.async_copy(h_spm.at[src_v.at[pl.ds(noff, CHUNK)]],
                                 buf.at[nslot], gsem.at[nslot])
            off = pl.multiple_of(j * CHUNK, 8)
            slot = lax.rem(j, NBUF)
            pltpu.make_async_copy(h_spm.at[src_v.at[pl.ds(off, CHUNK)]],
                                  buf.at[slot], gsem.at[slot]).wait()
            pltpu.sync_copy(buf.at[slot],
                            acc.at[dst_v.at[pl.ds(off, CHUNK)]], add=True)
            return carry

        lax.fori_loop(0, k, body, 0)
        plsc.subcore_barrier()

        pltpu.sync_copy(
            acc.at[pl.ds(row0, rows_per_tile)],
            out_hbm.at[cid, pl.ds(row0, rows_per_tile)])

    return agg_kernel


def _tc_matmul(x, w, n_pad, bm):
    n, kin = x.shape
    f = w.shape[1]
    assert n_pad % bm == 0 and bm % 8 == 0

    def body(x_ref, w_ref, o_ref):
        o_ref[...] = lax.dot_general(
            x_ref[...], w_ref[...], (((1,), (0,)), ((), ())),
            preferred_element_type=jnp.float32)

    return pl.pallas_call(
        body,
        grid=(n_pad // bm,),
        in_specs=[pl.BlockSpec((bm, kin), lambda i: (i, 0)),
                  pl.BlockSpec((kin, f), lambda i: (0, 0))],
        out_specs=pl.BlockSpec((bm, f), lambda i: (i, 0)),
        out_shape=jax.ShapeDtypeStruct((n_pad, f), jnp.float32),
    )(x, w)


def _tc_scale_flat(yf, degf):
    nf, _ = yf.shape

    def body(y_ref, d_ref, o_ref):
        d = d_ref[0, 0] + d_ref[1, 0]
        o_ref[...] = y_ref[...] * lax.rsqrt(jnp.maximum(d, 1.0))

    return pl.pallas_call(
        body,
        out_shape=jax.ShapeDtypeStruct((nf, 128), jnp.float32),
    )(yf, degf)


def _tc_relu_rescale(af, degf, b1):
    _, nf, _ = af.shape

    def body(a_ref, d_ref, b_ref, o_ref):
        a = a_ref[0] + a_ref[1]
        do = d_ref[0, 0] + d_ref[1, 0]
        di = d_ref[0, 1] + d_ref[1, 1]
        h = jnp.maximum(a * lax.rsqrt(jnp.maximum(di, 1.0)) + b_ref[...], 0.0)
        o_ref[...] = h * lax.rsqrt(jnp.maximum(do, 1.0))

    return pl.pallas_call(
        body,
        out_shape=jax.ShapeDtypeStruct((nf, 128), jnp.float32),
    )(af, degf, b1)


def _tc_final_flat(af, degf, w2bd, b2t):
    _, nf, _ = af.shape
    fo8 = w2bd.shape[1]

    def body(a_ref, d_ref, w_ref, b_ref, o_ref):
        a = a_ref[0] + a_ref[1]
        di = d_ref[0, 1] + d_ref[1, 1]
        hf = a * lax.rsqrt(jnp.maximum(di, 1.0))
        res = lax.dot_general(hf, w_ref[...], (((1,), (0,)), ((), ())),
                              preferred_element_type=jnp.float32)
        o_ref[...] = res + b_ref[...]

    return pl.pallas_call(
        body,
        out_shape=jax.ShapeDtypeStruct((nf, fo8), jnp.float32),
    )(af, degf, w2bd, b2t)


def kernel(features, edge_index, W1, b1, W2, b2):
    n, _ = features.shape
    e = edge_index.shape[1]
    n_pad = ((n + 8 * NS - 1) // (8 * NS)) * (8 * NS)
    nf = n_pad // 8
    trash = n_pad - 1

    k = -(-e // (NW * CHUNK))
    e_pad = NW * k * CHUNK
    pad = jnp.full((e_pad - e,), trash, jnp.int32)
    src_r = jnp.concatenate([edge_index[0], pad])
    dst_r = jnp.concatenate([edge_index[1], pad])

    deg = _make_degree_kernel(n_pad, k)(src_r, dst_r)
    degf = deg.reshape(NC, 2, nf, 128)
    y = _tc_matmul(features, W1, n_pad, bm=1264)
    h1f = _tc_scale_flat(y.reshape(nf, 128), degf)

    agg16 = _make_agg_kernel(n_pad, k, 16)
    b1t = jnp.tile(b1, 8).reshape(1, 128)
    a1 = agg16(h1f.reshape(n_pad, 16), src_r, dst_r)
    scaledf = _tc_relu_rescale(a1.reshape(NC, nf, 128), degf, b1t)
    a2 = agg16(scaledf.reshape(n_pad, 16), src_r, dst_r)

    fo = W2.shape[1]
    w2bd = jnp.kron(jnp.eye(8, dtype=jnp.float32), W2)
    b2t = jnp.tile(b2, 8).reshape(1, 8 * fo)
    res = _tc_final_flat(a2.reshape(NC, nf, 128), degf, w2bd, b2t)
    return res.reshape(n_pad, fo)[:n]


# speedup vs baseline: 1.8049x; 1.0289x over previous
import functools

import jax
import jax.numpy as jnp
from jax import lax
from jax.experimental import pallas as pl
from jax.experimental.pallas import tpu as pltpu
from jax.experimental.pallas import tpu_sc as plsc

NC = 2
NS = 16
NW = NC * NS
CHUNK = 128
NBUF = 4


def _mesh():
    return plsc.VectorSubcoreMesh(core_axis_name="c", subcore_axis_name="s")


def _make_degree_kernel(n_pad, e):
    rows_per_tile = n_pad // NS
    epw = e // NW
    kfull, rem = divmod(epw, CHUNK)
    assert rows_per_tile % 8 == 0 and e % NW == 0 and epw % 8 == 0
    assert rem % 8 == 0

    @functools.partial(
        pl.kernel,
        out_type=jax.ShapeDtypeStruct((NC, 2, n_pad * 16), jnp.float32),
        mesh=_mesh(),
        scratch_types=[
            pltpu.VMEM((epw,), jnp.int32),
            pltpu.VMEM((epw,), jnp.int32),
            pltpu.VMEM((CHUNK,), jnp.float32),
            pltpu.VMEM((rows_per_tile,), jnp.float32),
            pltpu.VMEM((rows_per_tile * 16,), jnp.float32),
            pltpu.VMEM((((rows_per_tile + 15) // 16) * 16,), jnp.float32),
            pltpu.VMEM_SHARED((n_pad,), jnp.float32),
            pltpu.VMEM_SHARED((n_pad,), jnp.float32),
            pltpu.SemaphoreType.DMA,
        ],
        compiler_params=pltpu.CompilerParams(use_tc_tiling_on_sc=False,
                                             needs_layout_passes=False),
    )
    def deg_kernel(src_hbm, dst_hbm, out_hbm,
                   src_v, dst_v, ones_v, dscal_v, exp_v, zscal_v,
                   acc_o, acc_i, sem):
        cid = lax.axis_index("c")
        sid = lax.axis_index("s")
        wid = sid * NC + cid
        row0 = sid * rows_per_tile

        def zfill(i, carry):
            zscal_v[pl.ds(i * 16, 16)] = jnp.zeros((16,), jnp.float32)
            return carry
        lax.fori_loop(0, zscal_v.shape[0] // 16, zfill, 0)

        def ofill(i, carry):
            ones_v[pl.ds(i * 16, 16)] = jnp.ones((16,), jnp.float32)
            return carry
        lax.fori_loop(0, CHUNK // 16, ofill, 0)

        pltpu.sync_copy(zscal_v.at[pl.ds(0, rows_per_tile)],
                        acc_o.at[pl.ds(row0, rows_per_tile)])
        pltpu.sync_copy(zscal_v.at[pl.ds(0, rows_per_tile)],
                        acc_i.at[pl.ds(row0, rows_per_tile)])
        pltpu.sync_copy(src_hbm.at[pl.ds(wid * epw, epw)], src_v)
        pltpu.sync_copy(dst_hbm.at[pl.ds(wid * epw, epw)], dst_v)
        plsc.subcore_barrier()

        def fire(j, carry):
            off = pl.multiple_of(j * CHUNK, 8)
            pltpu.async_copy(ones_v, acc_o.at[src_v.at[pl.ds(off, CHUNK)]],
                             sem, add=True)
            pltpu.async_copy(ones_v, acc_i.at[dst_v.at[pl.ds(off, CHUNK)]],
                             sem, add=True)
            return carry
        lax.fori_loop(0, kfull, fire, 0)
        if rem:
            t0 = kfull * CHUNK
            pltpu.async_copy(ones_v.at[pl.ds(0, rem)],
                             acc_o.at[src_v.at[pl.ds(t0, rem)]],
                             sem, add=True)
            pltpu.async_copy(ones_v.at[pl.ds(0, rem)],
                             acc_i.at[dst_v.at[pl.ds(t0, rem)]],
                             sem, add=True)

        def drain(j, carry):
            off = pl.multiple_of(j * CHUNK, 8)
            pltpu.make_async_copy(
                ones_v, acc_o.at[src_v.at[pl.ds(off, CHUNK)]], sem).wait()
            pltpu.make_async_copy(
                ones_v, acc_i.at[dst_v.at[pl.ds(off, CHUNK)]], sem).wait()
            return carry
        lax.fori_loop(0, kfull, drain, 0)
        if rem:
            t0 = kfull * CHUNK
            pltpu.make_async_copy(ones_v.at[pl.ds(0, rem)],
                                  acc_o.at[src_v.at[pl.ds(t0, rem)]],
                                  sem).wait()
            pltpu.make_async_copy(ones_v.at[pl.ds(0, rem)],
                                  acc_i.at[dst_v.at[pl.ds(t0, rem)]],
                                  sem).wait()
        plsc.subcore_barrier()

        for half, acc in ((0, acc_o), (1, acc_i)):
            pltpu.sync_copy(acc.at[pl.ds(row0, rows_per_tile)], dscal_v)

            def expand(r, carry):
                idx = jnp.full((16,), r, jnp.int32)
                off = pl.multiple_of(r * 16, 16)
                exp_v[pl.ds(off, 16)] = plsc.load_gather(dscal_v, [idx])
                return carry
            lax.fori_loop(0, rows_per_tile, expand, 0)
            pltpu.sync_copy(
                exp_v,
                out_hbm.at[cid, half,
                           pl.ds(row0 * 16, rows_per_tile * 16)])

    return deg_kernel


def _make_agg_kernel(n_pad, e, f):
    rows_per_tile = n_pad // NS
    epw = e // NW
    k, rem = divmod(epw, CHUNK)
    assert rows_per_tile % 8 == 0 and e % NW == 0 and epw % 8 == 0
    assert rem % 8 == 0

    @functools.partial(
        pl.kernel,
        out_type=jax.ShapeDtypeStruct((NC, n_pad, f), jnp.float32),
        mesh=_mesh(),
        scratch_types=[
            pltpu.VMEM((epw,), jnp.int32),
            pltpu.VMEM((epw,), jnp.int32),
            pltpu.VMEM((NBUF, CHUNK, f), jnp.float32),
            pltpu.VMEM((rows_per_tile, f), jnp.float32),
            pltpu.VMEM_SHARED((n_pad, f), jnp.float32),
            pltpu.VMEM_SHARED((n_pad, f), jnp.float32),
            pltpu.SemaphoreType.DMA((NBUF,)),
        ],
        compiler_params=pltpu.CompilerParams(use_tc_tiling_on_sc=False),
    )
    def agg_kernel(h_hbm, src_hbm, dst_hbm, out_hbm,
                   src_v, dst_v, buf, zbuf, acc, h_spm, gsem):
        cid = lax.axis_index("c")
        sid = lax.axis_index("s")
        wid = sid * NC + cid
        row0 = sid * rows_per_tile

        def zfill(i, carry):
            zbuf[i, :] = jnp.zeros((f,), jnp.float32)
            return carry
        lax.fori_loop(0, rows_per_tile, zfill, 0)
        pltpu.sync_copy(zbuf, acc.at[pl.ds(row0, rows_per_tile)])

        pltpu.sync_copy(h_hbm.at[pl.ds(row0, rows_per_tile)],
                        h_spm.at[pl.ds(row0, rows_per_tile)])
        pltpu.sync_copy(src_hbm.at[pl.ds(wid * epw, epw)], src_v)
        pltpu.sync_copy(dst_hbm.at[pl.ds(wid * epw, epw)], dst_v)
        plsc.subcore_barrier()

        pltpu.async_copy(h_spm.at[src_v.at[pl.ds(0, CHUNK)]],
                         buf.at[0], gsem.at[0])

        def body(j, carry):
            @pl.when(j + 1 < k)
            def _():
                noff = pl.multiple_of((j + 1) * CHUNK, 8)
                nslot = lax.rem(j + 1, NBUF)
                pltpu.async_copy(h_spm.at[src_v.at[pl.ds(noff, CHUNK)]],
                                 buf.at[nslot], gsem.at[nslot])
            off = pl.multiple_of(j * CHUNK, 8)
            slot = lax.rem(j, NBUF)
            pltpu.make_async_copy(h_spm.at[src_v.at[pl.ds(off, CHUNK)]],
                                  buf.at[slot], gsem.at[slot]).wait()
            pltpu.sync_copy(buf.at[slot],
                            acc.at[dst_v.at[pl.ds(off, CHUNK)]], add=True)
            return carry

        lax.fori_loop(0, k, body, 0)
        if rem:
            t0 = k * CHUNK
            pltpu.sync_copy(h_spm.at[src_v.at[pl.ds(t0, rem)]],
                            buf.at[k % NBUF, pl.ds(0, rem)])
            pltpu.sync_copy(buf.at[k % NBUF, pl.ds(0, rem)],
                            acc.at[dst_v.at[pl.ds(t0, rem)]], add=True)
        plsc.subcore_barrier()

        pltpu.sync_copy(
            acc.at[pl.ds(row0, rows_per_tile)],
            out_hbm.at[cid, pl.ds(row0, rows_per_tile)])

    return agg_kernel


def _tc_matmul(x, w, n_pad, bm):
    n, kin = x.shape
    f = w.shape[1]
    assert n_pad % bm == 0 and bm % 8 == 0

    def body(x_ref, w_ref, o_ref):
        o_ref[...] = lax.dot_general(
            x_ref[...], w_ref[...], (((1,), (0,)), ((), ())),
            preferred_element_type=jnp.float32)

    return pl.pallas_call(
        body,
        grid=(n_pad // bm,),
        in_specs=[pl.BlockSpec((bm, kin), lambda i: (i, 0)),
                  pl.BlockSpec((kin, f), lambda i: (0, 0))],
        out_specs=pl.BlockSpec((bm, f), lambda i: (i, 0)),
        out_shape=jax.ShapeDtypeStruct((n_pad, f), jnp.float32),
    )(x, w)


def _tc_scale_flat(yf, degf):
    nf, _ = yf.shape

    def body(y_ref, d_ref, o_ref):
        d = d_ref[0, 0] + d_ref[1, 0]
        o_ref[...] = y_ref[...] * lax.rsqrt(jnp.maximum(d, 1.0))

    return pl.pallas_call(
        body,
        out_shape=jax.ShapeDtypeStruct((nf, 128), jnp.float32),
    )(yf, degf)


def _tc_relu_rescale(af, degf, b1):
    _, nf, _ = af.shape

    def body(a_ref, d_ref, b_ref, o_ref):
        a = a_ref[0] + a_ref[1]
        do = d_ref[0, 0] + d_ref[1, 0]
        di = d_ref[0, 1] + d_ref[1, 1]
        h = jnp.maximum(a * lax.rsqrt(jnp.maximum(di, 1.0)) + b_ref[...], 0.0)
        o_ref[...] = h * lax.rsqrt(jnp.maximum(do, 1.0))

    return pl.pallas_call(
        body,
        out_shape=jax.ShapeDtypeStruct((nf, 128), jnp.float32),
    )(af, degf, b1)


def _tc_final_flat(af, degf, w2bd, b2t):
    _, nf, _ = af.shape
    fo8 = w2bd.shape[1]

    def body(a_ref, d_ref, w_ref, b_ref, o_ref):
        a = a_ref[0] + a_ref[1]
        di = d_ref[0, 1] + d_ref[1, 1]
        hf = a * lax.rsqrt(jnp.maximum(di, 1.0))
        res = lax.dot_general(hf, w_ref[...], (((1,), (0,)), ((), ())),
                              preferred_element_type=jnp.float32)
        o_ref[...] = res + b_ref[...]

    return pl.pallas_call(
        body,
        out_shape=jax.ShapeDtypeStruct((nf, fo8), jnp.float32),
    )(af, degf, w2bd, b2t)


def kernel(features, edge_index, W1, b1, W2, b2):
    n, _ = features.shape
    e = edge_index.shape[1]
    n_pad = ((n + 8 * NS - 1) // (8 * NS)) * (8 * NS)
    nf = n_pad // 8

    src_r = edge_index[0]
    dst_r = edge_index[1]

    deg = _make_degree_kernel(n_pad, e)(src_r, dst_r)
    degf = deg.reshape(NC, 2, nf, 128)
    y = _tc_matmul(features, W1, n_pad, bm=1264)
    h1f = _tc_scale_flat(y.reshape(nf, 128), degf)

    agg16 = _make_agg_kernel(n_pad, e, 16)
    b1t = jnp.tile(b1, 8).reshape(1, 128)
    a1 = agg16(h1f.reshape(n_pad, 16), src_r, dst_r)
    scaledf = _tc_relu_rescale(a1.reshape(NC, nf, 128), degf, b1t)
    a2 = agg16(scaledf.reshape(n_pad, 16), src_r, dst_r)

    fo = W2.shape[1]
    w2bd = jnp.kron(jnp.eye(8, dtype=jnp.float32), W2)
    b2t = jnp.tile(b2, 8).reshape(1, 8 * fo)
    res = _tc_final_flat(a2.reshape(NC, nf, 128), degf, w2bd, b2t)
    return res.reshape(n_pad, fo)[:n]


# speedup vs baseline: 1.8436x; 1.0214x over previous
import functools

import jax
import jax.numpy as jnp
from jax import lax
from jax.experimental import pallas as pl
from jax.experimental.pallas import tpu as pltpu
from jax.experimental.pallas import tpu_sc as plsc

NC = 2
NS = 16
NW = NC * NS
CHUNK = 128
NBUF = 4


def _mesh():
    return plsc.VectorSubcoreMesh(core_axis_name="c", subcore_axis_name="s")


def _make_degree_kernel(n_pad, e):
    rows_per_tile = n_pad // NS
    epw = e // NW
    kfull, rem = divmod(epw, CHUNK)
    assert rows_per_tile % 8 == 0 and e % NW == 0 and epw % 8 == 0
    assert rem % 8 == 0

    @functools.partial(
        pl.kernel,
        out_type=jax.ShapeDtypeStruct((NC, 2, n_pad * 16), jnp.float32),
        mesh=_mesh(),
        scratch_types=[
            pltpu.VMEM((epw,), jnp.int32),
            pltpu.VMEM((epw,), jnp.int32),
            pltpu.VMEM((CHUNK,), jnp.float32),
            pltpu.VMEM((rows_per_tile,), jnp.float32),
            pltpu.VMEM((rows_per_tile * 16,), jnp.float32),
            pltpu.VMEM((((rows_per_tile + 15) // 16) * 16,), jnp.float32),
            pltpu.VMEM_SHARED((n_pad,), jnp.float32),
            pltpu.VMEM_SHARED((n_pad,), jnp.float32),
            pltpu.SemaphoreType.DMA,
        ],
        compiler_params=pltpu.CompilerParams(use_tc_tiling_on_sc=False,
                                             needs_layout_passes=False),
    )
    def deg_kernel(edge_hbm, out_hbm,
                   src_v, dst_v, ones_v, dscal_v, exp_v, zscal_v,
                   acc_o, acc_i, sem):
        cid = lax.axis_index("c")
        sid = lax.axis_index("s")
        wid = sid * NC + cid
        row0 = sid * rows_per_tile

        def zfill(i, carry):
            zscal_v[pl.ds(i * 16, 16)] = jnp.zeros((16,), jnp.float32)
            return carry
        lax.fori_loop(0, zscal_v.shape[0] // 16, zfill, 0)

        def ofill(i, carry):
            ones_v[pl.ds(i * 16, 16)] = jnp.ones((16,), jnp.float32)
            return carry
        lax.fori_loop(0, CHUNK // 16, ofill, 0)

        pltpu.sync_copy(zscal_v.at[pl.ds(0, rows_per_tile)],
                        acc_o.at[pl.ds(row0, rows_per_tile)])
        pltpu.sync_copy(zscal_v.at[pl.ds(0, rows_per_tile)],
                        acc_i.at[pl.ds(row0, rows_per_tile)])
        pltpu.sync_copy(edge_hbm.at[pl.ds(wid * epw, epw)], src_v)
        pltpu.sync_copy(edge_hbm.at[pl.ds(e + wid * epw, epw)], dst_v)
        plsc.subcore_barrier()

        def fire(j, carry):
            off = pl.multiple_of(j * CHUNK, 8)
            pltpu.async_copy(ones_v, acc_o.at[src_v.at[pl.ds(off, CHUNK)]],
                             sem, add=True)
            pltpu.async_copy(ones_v, acc_i.at[dst_v.at[pl.ds(off, CHUNK)]],
                             sem, add=True)
            return carry
        lax.fori_loop(0, kfull, fire, 0)
        if rem:
            t0 = kfull * CHUNK
            pltpu.async_copy(ones_v.at[pl.ds(0, rem)],
                             acc_o.at[src_v.at[pl.ds(t0, rem)]],
                             sem, add=True)
            pltpu.async_copy(ones_v.at[pl.ds(0, rem)],
                             acc_i.at[dst_v.at[pl.ds(t0, rem)]],
                             sem, add=True)

        def drain(j, carry):
            off = pl.multiple_of(j * CHUNK, 8)
            pltpu.make_async_copy(
                ones_v, acc_o.at[src_v.at[pl.ds(off, CHUNK)]], sem).wait()
            pltpu.make_async_copy(
                ones_v, acc_i.at[dst_v.at[pl.ds(off, CHUNK)]], sem).wait()
            return carry
        lax.fori_loop(0, kfull, drain, 0)
        if rem:
            t0 = kfull * CHUNK
            pltpu.make_async_copy(ones_v.at[pl.ds(0, rem)],
                                  acc_o.at[src_v.at[pl.ds(t0, rem)]],
                                  sem).wait()
            pltpu.make_async_copy(ones_v.at[pl.ds(0, rem)],
                                  acc_i.at[dst_v.at[pl.ds(t0, rem)]],
                                  sem).wait()
        plsc.subcore_barrier()

        for half, acc in ((0, acc_o), (1, acc_i)):
            pltpu.sync_copy(acc.at[pl.ds(row0, rows_per_tile)], dscal_v)

            def expand(r, carry):
                idx = jnp.full((16,), r, jnp.int32)
                off = pl.multiple_of(r * 16, 16)
                exp_v[pl.ds(off, 16)] = plsc.load_gather(dscal_v, [idx])
                return carry
            lax.fori_loop(0, rows_per_tile, expand, 0)
            pltpu.sync_copy(
                exp_v,
                out_hbm.at[cid, half,
                           pl.ds(row0 * 16, rows_per_tile * 16)])

    return deg_kernel


def _make_agg_kernel(n_pad, e, f):
    rows_per_tile = n_pad // NS
    epw = e // NW
    k, rem = divmod(epw, CHUNK)
    assert rows_per_tile % 8 == 0 and e % NW == 0 and epw % 8 == 0
    assert rem % 8 == 0

    @functools.partial(
        pl.kernel,
        out_type=jax.ShapeDtypeStruct((NC, n_pad, f), jnp.float32),
        mesh=_mesh(),
        scratch_types=[
            pltpu.VMEM((epw,), jnp.int32),
            pltpu.VMEM((epw,), jnp.int32),
            pltpu.VMEM((NBUF, CHUNK, f), jnp.float32),
            pltpu.VMEM((rows_per_tile, f), jnp.float32),
            pltpu.VMEM_SHARED((n_pad, f), jnp.float32),
            pltpu.VMEM_SHARED((n_pad, f), jnp.float32),
            pltpu.SemaphoreType.DMA((NBUF,)),
        ],
        compiler_params=pltpu.CompilerParams(use_tc_tiling_on_sc=False),
    )
    def agg_kernel(h_hbm, edge_hbm, out_hbm,
                   src_v, dst_v, buf, zbuf, acc, h_spm, gsem):
        cid = lax.axis_index("c")
        sid = lax.axis_index("s")
        wid = sid * NC + cid
        row0 = sid * rows_per_tile

        def zfill(i, carry):
            zbuf[i, :] = jnp.zeros((f,), jnp.float32)
            return carry
        lax.fori_loop(0, rows_per_tile, zfill, 0)
        pltpu.sync_copy(zbuf, acc.at[pl.ds(row0, rows_per_tile)])

        pltpu.sync_copy(h_hbm.at[pl.ds(row0, rows_per_tile)],
                        h_spm.at[pl.ds(row0, rows_per_tile)])
        pltpu.sync_copy(edge_hbm.at[pl.ds(wid * epw, epw)], src_v)
        pltpu.sync_copy(edge_hbm.at[pl.ds(e + wid * epw, epw)], dst_v)
        plsc.subcore_barrier()

        pltpu.async_copy(h_spm.at[src_v.at[pl.ds(0, CHUNK)]],
                         buf.at[0], gsem.at[0])

        def body(j, carry):
            @pl.when(j + 1 < k)
            def _():
                noff = pl.multiple_of((j + 1) * CHUNK, 8)
                nslot = lax.rem(j + 1, NBUF)
                pltpu.async_copy(h_spm.at[src_v.at[pl.ds(noff, CHUNK)]],
                                 buf.at[nslot], gsem.at[nslot])
            off = pl.multiple_of(j * CHUNK, 8)
            slot = lax.rem(j, NBUF)
            pltpu.make_async_copy(h_spm.at[src_v.at[pl.ds(off, CHUNK)]],
                                  buf.at[slot], gsem.at[slot]).wait()
            pltpu.sync_copy(buf.at[slot],
                            acc.at[dst_v.at[pl.ds(off, CHUNK)]], add=True)
            return carry

        lax.fori_loop(0, k, body, 0)
        if rem:
            t0 = k * CHUNK
            pltpu.sync_copy(h_spm.at[src_v.at[pl.ds(t0, rem)]],
                            buf.at[k % NBUF, pl.ds(0, rem)])
            pltpu.sync_copy(buf.at[k % NBUF, pl.ds(0, rem)],
                            acc.at[dst_v.at[pl.ds(t0, rem)]], add=True)
        plsc.subcore_barrier()

        pltpu.sync_copy(
            acc.at[pl.ds(row0, rows_per_tile)],
            out_hbm.at[cid, pl.ds(row0, rows_per_tile)])

    return agg_kernel


def _tc_matmul(x, w, n_pad, bm):
    n, kin = x.shape
    f = w.shape[1]
    assert n_pad % bm == 0 and bm % 8 == 0

    def body(x_ref, w_ref, o_ref):
        o_ref[...] = lax.dot_general(
            x_ref[...], w_ref[...], (((1,), (0,)), ((), ())),
            preferred_element_type=jnp.float32)

    return pl.pallas_call(
        body,
        grid=(n_pad // bm,),
        in_specs=[pl.BlockSpec((bm, kin), lambda i: (i, 0)),
                  pl.BlockSpec((kin, f), lambda i: (0, 0))],
        out_specs=pl.BlockSpec((bm, f), lambda i: (i, 0)),
        out_shape=jax.ShapeDtypeStruct((n_pad, f), jnp.float32),
    )(x, w)


def _tc_scale_flat(yf, degf):
    nf, _ = yf.shape

    def body(y_ref, d_ref, o_ref):
        d = d_ref[0, 0] + d_ref[1, 0]
        o_ref[...] = y_ref[...] * lax.rsqrt(jnp.maximum(d, 1.0))

    return pl.pallas_call(
        body,
        out_shape=jax.ShapeDtypeStruct((nf, 128), jnp.float32),
    )(yf, degf)


def _tc_relu_rescale(af, degf, b1):
    _, nf, _ = af.shape

    def body(a_ref, d_ref, b_ref, o_ref):
        a = a_ref[0] + a_ref[1]
        do = d_ref[0, 0] + d_ref[1, 0]
        di = d_ref[0, 1] + d_ref[1, 1]
        h = jnp.maximum(a * lax.rsqrt(jnp.maximum(di, 1.0)) + b_ref[...], 0.0)
        o_ref[...] = h * lax.rsqrt(jnp.maximum(do, 1.0))

    return pl.pallas_call(
        body,
        out_shape=jax.ShapeDtypeStruct((nf, 128), jnp.float32),
    )(af, degf, b1)


def _tc_final_flat(af, degf, w2bd, b2t, n):
    fo8 = w2bd.shape[1]
    assert n % 8 == 0
    nr = n // 8

    def body(a_ref, d_ref, w_ref, b_ref, o_ref):
        a = a_ref[0, :nr, :] + a_ref[1, :nr, :]
        di = d_ref[0, 1, :nr, :] + d_ref[1, 1, :nr, :]
        hf = a * lax.rsqrt(jnp.maximum(di, 1.0))
        res = lax.dot_general(hf, w_ref[...], (((1,), (0,)), ((), ())),
                              preferred_element_type=jnp.float32)
        o_ref[...] = res + b_ref[...]

    return pl.pallas_call(
        body,
        out_shape=jax.ShapeDtypeStruct((nr, fo8), jnp.float32),
    )(af, degf, w2bd, b2t)


def kernel(features, edge_index, W1, b1, W2, b2):
    n, _ = features.shape
    e = edge_index.shape[1]
    n_pad = ((n + 8 * NS - 1) // (8 * NS)) * (8 * NS)
    nf = n_pad // 8

    edge_flat = edge_index.reshape(2 * e)

    deg = _make_degree_kernel(n_pad, e)(edge_flat)
    degf = deg.reshape(NC, 2, nf, 128)
    y = _tc_matmul(features, W1, n_pad, bm=632)
    h1f = _tc_scale_flat(y.reshape(nf, 128), degf)

    agg16 = _make_agg_kernel(n_pad, e, 16)
    b1t = jnp.tile(b1, 8).reshape(1, 128)
    a1 = agg16(h1f.reshape(n_pad, 16), edge_flat)
    scaledf = _tc_relu_rescale(a1.reshape(NC, nf, 128), degf, b1t)
    a2 = agg16(scaledf.reshape(n_pad, 16), edge_flat)

    fo = W2.shape[1]
    w2bd = jnp.kron(jnp.eye(8, dtype=jnp.float32), W2)
    b2t = jnp.tile(b2, 8).reshape(1, 8 * fo)
    res = _tc_final_flat(a2.reshape(NC, nf, 128), degf, w2bd, b2t, n)
    return res.reshape(n, fo)


# speedup vs baseline: 1.9491x; 1.0572x over previous
import functools

import jax
import jax.numpy as jnp
from jax import lax
from jax.experimental import pallas as pl
from jax.experimental.pallas import tpu as pltpu
from jax.experimental.pallas import tpu_sc as plsc

NC = 2
NS = 16
NW = NC * NS
CHUNK = 128
NBUF = 4


def _mesh():
    return plsc.VectorSubcoreMesh(core_axis_name="c", subcore_axis_name="s")


def _make_degree_kernel(n_pad, e):
    rows_per_tile = n_pad // NS
    epw = e // NW
    kfull, rem = divmod(epw, CHUNK)
    assert rows_per_tile % 8 == 0 and e % NW == 0 and epw % 8 == 0
    assert rem % 8 == 0

    @functools.partial(
        pl.kernel,
        out_type=jax.ShapeDtypeStruct((NC, 2, n_pad * 16), jnp.float32),
        mesh=_mesh(),
        scratch_types=[
            pltpu.VMEM((epw,), jnp.int32),
            pltpu.VMEM((epw,), jnp.int32),
            pltpu.VMEM((CHUNK,), jnp.float32),
            pltpu.VMEM((rows_per_tile,), jnp.float32),
            pltpu.VMEM((rows_per_tile * 16,), jnp.float32),
            pltpu.VMEM((((rows_per_tile + 15) // 16) * 16,), jnp.float32),
            pltpu.VMEM_SHARED((n_pad,), jnp.float32),
            pltpu.VMEM_SHARED((n_pad,), jnp.float32),
            pltpu.SemaphoreType.DMA,
        ],
        compiler_params=pltpu.CompilerParams(use_tc_tiling_on_sc=False,
                                             needs_layout_passes=False),
    )
    def deg_kernel(edge_hbm, out_hbm,
                   src_v, dst_v, ones_v, dscal_v, exp_v, zscal_v,
                   acc_o, acc_i, sem):
        cid = lax.axis_index("c")
        sid = lax.axis_index("s")
        wid = sid * NC + cid
        row0 = sid * rows_per_tile

        def zfill(i, carry):
            zscal_v[pl.ds(i * 16, 16)] = jnp.zeros((16,), jnp.float32)
            return carry
        lax.fori_loop(0, zscal_v.shape[0] // 16, zfill, 0)

        def ofill(i, carry):
            ones_v[pl.ds(i * 16, 16)] = jnp.ones((16,), jnp.float32)
            return carry
        lax.fori_loop(0, CHUNK // 16, ofill, 0)

        pltpu.sync_copy(zscal_v.at[pl.ds(0, rows_per_tile)],
                        acc_o.at[pl.ds(row0, rows_per_tile)])
        pltpu.sync_copy(zscal_v.at[pl.ds(0, rows_per_tile)],
                        acc_i.at[pl.ds(row0, rows_per_tile)])
        pltpu.sync_copy(edge_hbm.at[pl.ds(wid * epw, epw)], src_v)
        pltpu.sync_copy(edge_hbm.at[pl.ds(e + wid * epw, epw)], dst_v)
        plsc.subcore_barrier()

        def fire(j, carry):
            off = pl.multiple_of(j * CHUNK, 8)
            pltpu.async_copy(ones_v, acc_o.at[src_v.at[pl.ds(off, CHUNK)]],
                             sem, add=True)
            pltpu.async_copy(ones_v, acc_i.at[dst_v.at[pl.ds(off, CHUNK)]],
                             sem, add=True)
            return carry
        lax.fori_loop(0, kfull, fire, 0)
        if rem:
            t0 = kfull * CHUNK
            pltpu.async_copy(ones_v.at[pl.ds(0, rem)],
                             acc_o.at[src_v.at[pl.ds(t0, rem)]],
                             sem, add=True)
            pltpu.async_copy(ones_v.at[pl.ds(0, rem)],
                             acc_i.at[dst_v.at[pl.ds(t0, rem)]],
                             sem, add=True)

        def drain(j, carry):
            off = pl.multiple_of(j * CHUNK, 8)
            pltpu.make_async_copy(
                ones_v, acc_o.at[src_v.at[pl.ds(off, CHUNK)]], sem).wait()
            pltpu.make_async_copy(
                ones_v, acc_i.at[dst_v.at[pl.ds(off, CHUNK)]], sem).wait()
            return carry
        lax.fori_loop(0, kfull, drain, 0)
        if rem:
            t0 = kfull * CHUNK
            pltpu.make_async_copy(ones_v.at[pl.ds(0, rem)],
                                  acc_o.at[src_v.at[pl.ds(t0, rem)]],
                                  sem).wait()
            pltpu.make_async_copy(ones_v.at[pl.ds(0, rem)],
                                  acc_i.at[dst_v.at[pl.ds(t0, rem)]],
                                  sem).wait()
        plsc.subcore_barrier()

        for half, acc in ((0, acc_o), (1, acc_i)):
            pltpu.sync_copy(acc.at[pl.ds(row0, rows_per_tile)], dscal_v)

            def expand(r, carry):
                idx = jnp.full((16,), r, jnp.int32)
                off = pl.multiple_of(r * 16, 16)
                exp_v[pl.ds(off, 16)] = plsc.load_gather(dscal_v, [idx])
                return carry
            lax.fori_loop(0, rows_per_tile, expand, 0)
            pltpu.sync_copy(
                exp_v,
                out_hbm.at[cid, half,
                           pl.ds(row0 * 16, rows_per_tile * 16)])

    return deg_kernel


def _make_agg_kernel(n_pad, e, f):
    rows_per_tile = n_pad // NS
    epw = e // NW
    k, rem = divmod(epw, CHUNK)
    assert rows_per_tile % 8 == 0 and e % NW == 0 and epw % 8 == 0
    assert rem % 8 == 0

    @functools.partial(
        pl.kernel,
        out_type=jax.ShapeDtypeStruct((NC, n_pad, f), jnp.float32),
        mesh=_mesh(),
        scratch_types=[
            pltpu.VMEM((epw,), jnp.int32),
            pltpu.VMEM((epw,), jnp.int32),
            pltpu.VMEM((NBUF, CHUNK, f), jnp.float32),
            pltpu.VMEM((rows_per_tile, f), jnp.float32),
            pltpu.VMEM_SHARED((n_pad, f), jnp.float32),
            pltpu.VMEM_SHARED((n_pad, f), jnp.float32),
            pltpu.SemaphoreType.DMA((NBUF,)),
        ],
        compiler_params=pltpu.CompilerParams(use_tc_tiling_on_sc=False),
    )
    def agg_kernel(h_hbm, edge_hbm, out_hbm,
                   src_v, dst_v, buf, zbuf, acc, h_spm, gsem):
        cid = lax.axis_index("c")
        sid = lax.axis_index("s")
        wid = sid * NC + cid
        row0 = sid * rows_per_tile

        def zfill(i, carry):
            zbuf[i, :] = jnp.zeros((f,), jnp.float32)
            return carry
        lax.fori_loop(0, rows_per_tile, zfill, 0)
        pltpu.sync_copy(zbuf, acc.at[pl.ds(row0, rows_per_tile)])

        pltpu.sync_copy(h_hbm.at[pl.ds(row0, rows_per_tile)],
                        h_spm.at[pl.ds(row0, rows_per_tile)])
        pltpu.sync_copy(edge_hbm.at[pl.ds(wid * epw, epw)], src_v)
        pltpu.sync_copy(edge_hbm.at[pl.ds(e + wid * epw, epw)], dst_v)
        plsc.subcore_barrier()

        pltpu.async_copy(h_spm.at[src_v.at[pl.ds(0, CHUNK)]],
                         buf.at[0], gsem.at[0])

        def body(j, carry):
            @pl.when(j + 1 < k)
            def _():
                noff = pl.multiple_of((j + 1) * CHUNK, 8)
                nslot = lax.rem(j + 1, NBUF)
                pltpu.async_copy(h_spm.at[src_v.at[pl.ds(noff, CHUNK)]],
                                 buf.at[nslot], gsem.at[nslot])
            off = pl.multiple_of(j * CHUNK, 8)
            slot = lax.rem(j, NBUF)
            pltpu.make_async_copy(h_spm.at[src_v.at[pl.ds(off, CHUNK)]],
                                  buf.at[slot], gsem.at[slot]).wait()
            pltpu.sync_copy(buf.at[slot],
                            acc.at[dst_v.at[pl.ds(off, CHUNK)]], add=True)
            return carry

        lax.fori_loop(0, k, body, 0)
        if rem:
            t0 = k * CHUNK
            pltpu.sync_copy(h_spm.at[src_v.at[pl.ds(t0, rem)]],
                            buf.at[k % NBUF, pl.ds(0, rem)])
            pltpu.sync_copy(buf.at[k % NBUF, pl.ds(0, rem)],
                            acc.at[dst_v.at[pl.ds(t0, rem)]], add=True)
        plsc.subcore_barrier()

        pltpu.sync_copy(
            acc.at[pl.ds(row0, rows_per_tile)],
            out_hbm.at[cid, pl.ds(row0, rows_per_tile)])

    return agg_kernel


def _tc_matmul(x, w, n_pad, bm):
    n, kin = x.shape
    f = w.shape[1]
    assert n_pad % bm == 0 and bm % 8 == 0

    def body(x_ref, w_ref, o_ref):
        o_ref[...] = lax.dot_general(
            x_ref[...], w_ref[...], (((1,), (0,)), ((), ())),
            preferred_element_type=jnp.float32)

    return pl.pallas_call(
        body,
        grid=(n_pad // bm,),
        in_specs=[pl.BlockSpec((bm, kin), lambda i: (i, 0)),
                  pl.BlockSpec((kin, f), lambda i: (0, 0))],
        out_specs=pl.BlockSpec((bm, f), lambda i: (i, 0)),
        out_shape=jax.ShapeDtypeStruct((n_pad, f), jnp.float32),
    )(x, w)


def _tc_edge_flatten(edge_index):
    two, e = edge_index.shape

    def body(x_ref, o_ref):
        o_ref[pl.ds(0, e)] = x_ref[0, :]
        o_ref[pl.ds(e, e)] = x_ref[1, :]

    return pl.pallas_call(
        body,
        out_shape=jax.ShapeDtypeStruct((two * e,), jnp.int32),
    )(edge_index)


def _tc_scale_flat(yf, degf):
    nf, _ = yf.shape

    def body(y_ref, d_ref, o_ref):
        d = d_ref[0, 0] + d_ref[1, 0]
        o_ref[...] = y_ref[...] * lax.rsqrt(jnp.maximum(d, 1.0))

    return pl.pallas_call(
        body,
        out_shape=jax.ShapeDtypeStruct((nf, 128), jnp.float32),
    )(yf, degf)


def _tc_relu_rescale(af, degf, b1):
    _, nf, _ = af.shape

    def body(a_ref, d_ref, b_ref, o_ref):
        a = a_ref[0] + a_ref[1]
        do = d_ref[0, 0] + d_ref[1, 0]
        di = d_ref[0, 1] + d_ref[1, 1]
        h = jnp.maximum(a * lax.rsqrt(jnp.maximum(di, 1.0)) + b_ref[...], 0.0)
        o_ref[...] = h * lax.rsqrt(jnp.maximum(do, 1.0))

    return pl.pallas_call(
        body,
        out_shape=jax.ShapeDtypeStruct((nf, 128), jnp.float32),
    )(af, degf, b1)


def _tc_final_flat(af, degf, w2bd, b2t, n):
    fo8 = w2bd.shape[1]
    assert n % 8 == 0
    nr = n // 8

    def body(a_ref, d_ref, w_ref, b_ref, o_ref):
        a = a_ref[0, :nr, :] + a_ref[1, :nr, :]
        di = d_ref[0, 1, :nr, :] + d_ref[1, 1, :nr, :]
        hf = a * lax.rsqrt(jnp.maximum(di, 1.0))
        res = lax.dot_general(hf, w_ref[...], (((1,), (0,)), ((), ())),
                              preferred_element_type=jnp.float32)
        o_ref[...] = res + b_ref[...]

    return pl.pallas_call(
        body,
        out_shape=jax.ShapeDtypeStruct((nr, fo8), jnp.float32),
    )(af, degf, w2bd, b2t)


def kernel(features, edge_index, W1, b1, W2, b2):
    n, _ = features.shape
    e = edge_index.shape[1]
    n_pad = ((n + 8 * NS - 1) // (8 * NS)) * (8 * NS)
    nf = n_pad // 8

    edge_flat = _tc_edge_flatten(edge_index)

    deg = _make_degree_kernel(n_pad, e)(edge_flat)
    degf = deg.reshape(NC, 2, nf, 128)
    y = _tc_matmul(features, W1, n_pad, bm=1264)
    h1f = _tc_scale_flat(y.reshape(nf, 128), degf)

    agg16 = _make_agg_kernel(n_pad, e, 16)
    b1t = jnp.tile(b1, 8).reshape(1, 128)
    a1 = agg16(h1f.reshape(n_pad, 16), edge_flat)
    scaledf = _tc_relu_rescale(a1.reshape(NC, nf, 128), degf, b1t)
    a2 = agg16(scaledf.reshape(n_pad, 16), edge_flat)

    fo = W2.shape[1]
    w2bd = jnp.kron(jnp.eye(8, dtype=jnp.float32), W2)
    b2t = jnp.tile(b2, 8).reshape(1, 8 * fo)
    res = _tc_final_flat(a2.reshape(NC, nf, 128), degf, w2bd, b2t, n)
    return res.reshape(n, fo)
